# Initial kernel scaffold; baseline (speedup 1.0000x reference)
import jax
import jax.numpy as jnp
from jax.experimental import pallas as pl


def kernel(hidden_states, W_gate, Wg_e, Wu_e, Wd_e, Wg_s, Wu_s, Wd_s):
    raise NotImplementedError("write your pallas kernel here")


# speedup vs baseline: 1.3167x; 1.3167x over previous
import functools

import jax
import jax.numpy as jnp
from jax import lax
from jax.experimental import pallas as pl
from jax.experimental.pallas import tpu as pltpu
from jax.experimental.pallas import tpu_sc as plsc

S = 2048
H = 2048
E = 8
K = 2
I = 1408
IS = 2 * I
SK = S * K
T = 256
G = SK // T + E - 1
R = G * T

_C = 512
F32 = jnp.float32
I32 = jnp.int32
BF16 = jnp.bfloat16


def _router_body(x_ref, wg_ref, idx_ref, w_ref, pos_ref, meta_ref):
    x = x_ref[...]
    logits = lax.dot_general(x, wg_ref[...], (((1,), (1,)), ((), ())),
                             preferred_element_type=F32)
    m = jnp.max(logits, axis=1, keepdims=True)
    p = jnp.exp(logits - m)
    scores = p / jnp.sum(p, axis=1, keepdims=True)

    i8 = lax.broadcasted_iota(I32, (S, E), 1).astype(F32)
    w0 = jnp.max(scores, axis=1, keepdims=True)
    idx0 = jnp.min(jnp.where(scores == w0, i8, float(E)), axis=1,
                   keepdims=True)
    masked = jnp.where(i8 == idx0, -1.0, scores)
    w1 = jnp.max(masked, axis=1, keepdims=True)
    idx1 = jnp.min(jnp.where(masked == w1, i8, float(E)), axis=1,
                   keepdims=True)

    idx_ref[...] = jnp.concatenate([idx0.astype(I32), idx1.astype(I32)],
                                   axis=1)
    w_cat = jnp.concatenate([w0, w1], axis=0)
    w_ref[...] = w_cat * jnp.ones((1, 16), F32)

    oh = jnp.concatenate([(i8 == idx0).astype(F32),
                          (i8 == idx1).astype(F32)], axis=0)

    tri = (lax.broadcasted_iota(I32, (_C, _C), 0) >=
           lax.broadcasted_iota(I32, (_C, _C), 1)).astype(F32)
    carry = jnp.zeros((1, E), F32)
    chunks = []
    for c in range(SK // _C):
        seg = oh[c * _C:(c + 1) * _C]
        incl_c = lax.dot_general(tri, seg, (((1,), (0,)), ((), ())),
                                 preferred_element_type=F32) + carry
        chunks.append(incl_c)
        carry = incl_c[_C - 1:_C, :]
    incl = jnp.concatenate(chunks, axis=0)
    rank = incl - oh

    counts = incl[SK - 1:SK, :]
    ci = counts.astype(I32)
    pc = jnp.bitwise_and(ci + (T - 1), -T)
    pcf = pc.astype(F32)
    upper = (lax.broadcasted_iota(I32, (E, E), 0) <
             lax.broadcasted_iota(I32, (E, E), 1)).astype(F32)
    aligned = lax.dot_general(pcf, upper, (((1,), (0,)), ((), ())),
                              preferred_element_type=F32)
    posf = jnp.sum(oh * (aligned + rank), axis=1, keepdims=True)
    pos_ref[...] = posf.astype(I32)

    ends = aligned + pcf
    gvals = lax.broadcasted_iota(I32, (G, 1), 0).astype(F32) * float(T)
    be = jnp.sum((gvals >= ends).astype(F32), axis=1, keepdims=True)
    eid = lax.broadcasted_iota(I32, (1, E), 1).astype(F32)
    last_e = jnp.max(jnp.where(pcf > 0, eid, 0.0), axis=1, keepdims=True)
    bec = jnp.minimum(be, last_e)
    used = jnp.sum(pcf, axis=1, keepdims=True) * (1.0 / T)
    meta_ref[...] = jnp.concatenate([used, bec], axis=0).astype(I32)


def _router(x, w_gate):
    return pl.pallas_call(
        _router_body,
        out_shape=[
            jax.ShapeDtypeStruct((S, K), I32),
            jax.ShapeDtypeStruct((SK, 16), F32),
            jax.ShapeDtypeStruct((SK, 1), I32),
            jax.ShapeDtypeStruct((G + 1, 1), I32),
        ],
    )(x, w_gate)


_NW = 32
_PPW = SK // _NW
_PB = 32


def _builder(x, pos):
    mesh = plsc.VectorSubcoreMesh(core_axis_name="c", subcore_axis_name="s")

    @functools.partial(
        pl.kernel,
        mesh=mesh,
        out_type=jax.ShapeDtypeStruct((R, H), F32),
        scratch_types=[
            pltpu.VMEM((_PB,), I32),
            pltpu.VMEM((_PB,), I32),
            pltpu.VMEM((_PB, H), F32),
            pltpu.SemaphoreType.DMA,
            pltpu.SemaphoreType.DMA,
        ],
    )
    def body(x_hbm, pos_hbm, xs_hbm, posv, tokv, rows, sem1, sem2):
        wid = lax.axis_index("s") * 2 + lax.axis_index("c")
        sub = jnp.where(wid >= _NW // 2, S, 0)
        for b in range(_PPW // _PB):
            base = wid * _PPW + b * _PB
            pltpu.sync_copy(pos_hbm.at[pl.ds(base, _PB)], posv)
            for h in range(_PB // 16):
                tokv[pl.ds(h * 16, 16)] = (
                    base - sub + h * 16
                    + lax.broadcasted_iota(I32, (16,), 0))
            pltpu.async_copy(x_hbm.at[tokv], rows, sem1).wait()
            pltpu.async_copy(rows, xs_hbm.at[posv], sem2).wait()

    return body(x, pos)


def _grouped_body(m_ref, xs_ref, wg_ref, wu_ref, wd_ref, o_ref):
    g = pl.program_id(0)

    @pl.when(g < m_ref[0])
    def _():
        xb = xs_ref[...].astype(BF16)
        gate = lax.dot_general(xb, wg_ref[0], (((1,), (1,)), ((), ())),
                               preferred_element_type=F32)
        up = lax.dot_general(xb, wu_ref[0], (((1,), (1,)), ((), ())),
                             preferred_element_type=F32)
        h = jax.nn.silu(gate) * up
        o_ref[...] = lax.dot_general(h.astype(BF16), wd_ref[0],
                                     (((1,), (1,)), ((), ())),
                                     preferred_element_type=F32)


def _grouped(meta, xs, wg_e, wu_e, wd_e):
    grid_spec = pltpu.PrefetchScalarGridSpec(
        num_scalar_prefetch=1,
        grid=(G,),
        in_specs=[
            pl.BlockSpec((T, H), lambda g, m: (jnp.minimum(g, m[0] - 1), 0)),
            pl.BlockSpec((1, I, H), lambda g, m: (m[1 + g], 0, 0)),
            pl.BlockSpec((1, I, H), lambda g, m: (m[1 + g], 0, 0)),
            pl.BlockSpec((1, H, I), lambda g, m: (m[1 + g], 0, 0)),
        ],
        out_specs=pl.BlockSpec((T, H), lambda g, m: (g, 0)),
    )
    return pl.pallas_call(
        _grouped_body,
        grid_spec=grid_spec,
        out_shape=jax.ShapeDtypeStruct((R, H), F32),
    )(meta, xs, wg_e, wu_e, wd_e)


_TPW = S // _NW
_TB = 16


def _combine(ys, pos, wflat):
    mesh = plsc.VectorSubcoreMesh(core_axis_name="c", subcore_axis_name="s")

    @functools.partial(
        pl.kernel,
        mesh=mesh,
        out_type=jax.ShapeDtypeStruct((S, H), F32),
        scratch_types=[
            pltpu.VMEM((2 * _TB,), I32),
            pltpu.VMEM((2 * _TB, 16), F32),
            pltpu.VMEM((2 * _TB, H), F32),
            pltpu.VMEM((_TB, H), F32),
            pltpu.SemaphoreType.DMA,
        ],
    )
    def body(ys_hbm, pos_hbm, w_hbm, out_hbm, idxv, wv, yrows, orows, sem):
        wid = lax.axis_index("s") * 2 + lax.axis_index("c")
        for b in range(_TPW // _TB):
            t0 = wid * _TPW + b * _TB
            pltpu.sync_copy(pos_hbm.at[pl.ds(t0, _TB)],
                            idxv.at[pl.ds(0, _TB)])
            pltpu.sync_copy(pos_hbm.at[pl.ds(S + t0, _TB)],
                            idxv.at[pl.ds(_TB, _TB)])
            pltpu.sync_copy(w_hbm.at[pl.ds(t0, _TB)], wv.at[pl.ds(0, _TB)])
            pltpu.sync_copy(w_hbm.at[pl.ds(S + t0, _TB)],
                            wv.at[pl.ds(_TB, _TB)])
            pltpu.async_copy(ys_hbm.at[idxv], yrows, sem).wait()
            for j in range(_TB):
                wa = wv[j, pl.ds(0, 16)]
                wb = wv[_TB + j, pl.ds(0, 16)]

                def cbody(c, _):
                    sl = pl.ds(c * 16, 16)
                    orows[j, sl] = (wa * yrows[j, sl]
                                    + wb * yrows[_TB + j, sl])
                    return 0
                lax.fori_loop(0, H // 16, cbody, 0)
            pltpu.sync_copy(orows, out_hbm.at[pl.ds(t0, _TB)])

    return body(ys, pos, wflat)


def _shared_body(x_ref, wg_ref, wu_ref, wd_ref, r_ref, o_ref):
    xb = x_ref[...].astype(BF16)
    gate = lax.dot_general(xb, wg_ref[...], (((1,), (1,)), ((), ())),
                           preferred_element_type=F32)
    up = lax.dot_general(xb, wu_ref[...], (((1,), (1,)), ((), ())),
                         preferred_element_type=F32)
    h = (jax.nn.silu(gate) * up).astype(BF16)
    o_ref[...] = lax.dot_general(h, wd_ref[...], (((1,), (1,)), ((), ())),
                                 preferred_element_type=F32) + r_ref[...]


_TS = 256


def _shared(x, wg_s, wu_s, wd_s, routed):
    return pl.pallas_call(
        _shared_body,
        grid=(S // _TS,),
        in_specs=[
            pl.BlockSpec((_TS, H), lambda g: (g, 0)),
            pl.BlockSpec((IS, H), lambda g: (0, 0)),
            pl.BlockSpec((IS, H), lambda g: (0, 0)),
            pl.BlockSpec((H, IS), lambda g: (0, 0)),
            pl.BlockSpec((_TS, H), lambda g: (g, 0)),
        ],
        out_specs=pl.BlockSpec((_TS, H), lambda g: (g, 0)),
        out_shape=jax.ShapeDtypeStruct((S, H), F32),
    )(x, wg_s, wu_s, wd_s, routed)


def kernel(hidden_states, W_gate, Wg_e, Wu_e, Wd_e, Wg_s, Wu_s, Wd_s):
    x = hidden_states.reshape(S, H)
    o_idx, o_w, o_pos, o_meta = _router(x, W_gate)
    pos = o_pos.reshape(SK)
    meta = o_meta.reshape(G + 1)
    xs = _builder(x, pos)
    ys = _grouped(meta, xs,
                  Wg_e.astype(BF16), Wu_e.astype(BF16), Wd_e.astype(BF16))
    routed = _combine(ys, pos, o_w)
    out = _shared(x, Wg_s.astype(BF16), Wu_s.astype(BF16), Wd_s.astype(BF16),
                  routed)
    return out.reshape(1, S, H), o_idx.reshape(1, S, K)
